# baseline (speedup 1.0000x reference)
import functools

import jax
import jax.numpy as jnp
from jax.experimental import pallas as pl
from jax.experimental.pallas import tpu as pltpu

_BR = 128
_L = 128


def _cumsum_kernel(w_ref, x_ref, o_ref, *, br, n, l):
    w = w_ref[...]
    carry = jnp.zeros((br, l), dtype=jnp.float32)
    for c in range(n // l):
        xc = x_ref[:, c * l : (c + 1) * l]
        y = jnp.dot(xc, w, preferred_element_type=jnp.float32)
        o_ref[:, c * l : (c + 1) * l] = y[:, :l] + carry
        carry = carry + y[:, l:]


@jax.jit
def kernel(x):
    m, n = x.shape
    tri = jnp.triu(jnp.ones((_L, _L), dtype=jnp.float32))
    w = jnp.concatenate([tri, jnp.ones((_L, _L), dtype=jnp.float32)], axis=1)
    return pl.pallas_call(
        functools.partial(_cumsum_kernel, br=_BR, n=n, l=_L),
        grid=(m // _BR,),
        in_specs=[
            pl.BlockSpec((_L, 2 * _L), lambda i: (0, 0)),
            pl.BlockSpec((_BR, n), lambda i: (i, 0)),
        ],
        out_specs=pl.BlockSpec((_BR, n), lambda i: (i, 0)),
        out_shape=jax.ShapeDtypeStruct((m, n), x.dtype),
        compiler_params=pltpu.CompilerParams(
            dimension_semantics=("parallel",)
        ),
    )(w, x)


# speedup vs baseline: 1.0322x; 1.0322x over previous
import functools

import jax
import jax.numpy as jnp
from jax.experimental import pallas as pl
from jax.experimental.pallas import tpu as pltpu

_BR = 256
_L = 128


def _cumsum_kernel(w_ref, x_ref, o_ref, *, br, n, l):
    w = w_ref[...]
    carry = jnp.zeros((br, l), dtype=jnp.float32)
    for c in range(n // l):
        xc = x_ref[:, c * l : (c + 1) * l]
        y = jnp.dot(xc, w, preferred_element_type=jnp.float32)
        o_ref[:, c * l : (c + 1) * l] = y[:, :l] + carry
        carry = carry + y[:, l:]


@jax.jit
def kernel(x):
    m, n = x.shape
    tri = jnp.triu(jnp.ones((_L, _L), dtype=jnp.float32))
    w = jnp.concatenate([tri, jnp.ones((_L, _L), dtype=jnp.float32)], axis=1)
    return pl.pallas_call(
        functools.partial(_cumsum_kernel, br=_BR, n=n, l=_L),
        grid=(m // _BR,),
        in_specs=[
            pl.BlockSpec((_L, 2 * _L), lambda i: (0, 0)),
            pl.BlockSpec((_BR, n), lambda i: (i, 0)),
        ],
        out_specs=pl.BlockSpec((_BR, n), lambda i: (i, 0)),
        out_shape=jax.ShapeDtypeStruct((m, n), x.dtype),
        compiler_params=pltpu.CompilerParams(
            dimension_semantics=("parallel",)
        ),
    )(w, x)
